# Initial kernel scaffold; baseline (speedup 1.0000x reference)
import jax
import jax.numpy as jnp
from jax.experimental import pallas as pl


def kernel(x, W_in, A_log, W_x, W_dt, b_dt, D_param, W_out, gamma, beta):
    raise NotImplementedError("write your pallas kernel here")


# speedup vs baseline: 15.9657x; 15.9657x over previous
import functools
import math

import jax
import jax.numpy as jnp
from jax.experimental import pallas as pl
from jax.experimental.pallas import tpu as pltpu

B, L, D = 8, 256, 384
E, N, R = 768, 16, 24
HW = 16
EPS = 1e-5
FILL_CHUNK = 8


def _vssm_kernel(x_ref, w_in_x_ref, w_in_z_ref, wx_dt_ref, wx_b_ref,
                 wx_c_ref, w_dt_t_ref, b_dt_ref, a_log_t_ref, d_param_ref,
                 w_out_t_ref, gamma_ref, beta_ref, o_ref,
                 xin_s, z_s, dt_s, bp_s, cp_s, a_s,
                 abar_s, bbar_s, cful_s, yacc_s, h_s):
    xb = x_ref[0]

    xin = jnp.dot(xb, w_in_x_ref[...], preferred_element_type=jnp.float32)
    xin_s[...] = xin
    z_s[...] = jnp.dot(xb, w_in_z_ref[...], preferred_element_type=jnp.float32)
    dt_in = jnp.dot(xin, wx_dt_ref[...], preferred_element_type=jnp.float32)
    bp_s[...] = jnp.dot(xin, wx_b_ref[...], preferred_element_type=jnp.float32)
    cp_s[...] = jnp.dot(xin, wx_c_ref[...], preferred_element_type=jnp.float32)
    dt_raw = jnp.dot(dt_in, w_dt_t_ref[...],
                     preferred_element_type=jnp.float32) + b_dt_ref[...]
    dt_s[...] = jnp.maximum(dt_raw, 0.0) + jnp.log1p(jnp.exp(-jnp.abs(dt_raw)))
    a_s[...] = -jnp.exp(a_log_t_ref[...])

    def fill_body(i, _):
        sl = pl.ds(i * FILL_CHUNK, FILL_CHUNK)
        dt_c = dt_s[sl, :]
        xin_c = xin_s[sl, :]
        bp_c = bp_s[sl, :]
        cp_c = cp_s[sl, :]
        a_v = a_s[...]
        abar_s[sl] = jnp.exp(dt_c[:, None, :] * a_v[None, :, :])
        bbar_s[sl] = (dt_c * xin_c)[:, None, :] * bp_c[:, :, None]
        cful_s[sl] = jnp.broadcast_to(cp_c[:, :, None], (FILL_CHUNK, N, E))
        return 0

    jax.lax.fori_loop(0, L // FILL_CHUNK, fill_body, 0)

    h_s[...] = jnp.zeros_like(h_s)
    yacc_s[...] = jnp.zeros_like(yacc_s)

    def scan_body(t, _):
        u = (L - 1) - t
        ls = (t, u,
              ((t & (HW - 1)) << 4) | (t >> 4),
              ((u & (HW - 1)) << 4) | (u >> 4))
        for d in range(4):
            l = ls[d]
            a = abar_s[l]
            bb = bbar_s[l]
            cc = cful_s[l]
            hn = a * h_s[d] + bb
            h_s[d] = hn
            yacc_s[pl.ds(l, 1), :] += jnp.sum(hn * cc, axis=0, keepdims=True)
        return 0

    jax.lax.fori_loop(0, L, scan_body, 0)

    z = z_s[...]
    sig = 1.0 / (1.0 + jnp.exp(-z))
    y = yacc_s[...] * 0.25 * (z * sig) + xin_s[...] * d_param_ref[...]
    out = jnp.dot(y, w_out_t_ref[...], preferred_element_type=jnp.float32) + xb
    mu = jnp.mean(out, axis=-1, keepdims=True)
    xc = out - mu
    var = jnp.mean(xc * xc, axis=-1, keepdims=True)
    o_ref[0] = xc * jax.lax.rsqrt(var + EPS) * gamma_ref[...] + beta_ref[...]


@jax.jit
def kernel(x, W_in, A_log, W_x, W_dt, b_dt, D_param, W_out, gamma, beta):
    w_in_x = W_in[:E].T
    w_in_z = W_in[E:].T
    wx_dt = W_x[:R].T
    wx_b = W_x[R:R + N].T
    wx_c = W_x[R + N:].T
    w_dt_t = W_dt.T
    a_log_t = A_log.T
    w_out_t = W_out.T

    whole = lambda s: pl.BlockSpec(s, lambda b: tuple(0 for _ in s))
    f32 = jnp.float32
    return pl.pallas_call(
        _vssm_kernel,
        out_shape=jax.ShapeDtypeStruct((B, L, D), f32),
        grid=(B,),
        in_specs=[
            pl.BlockSpec((1, L, D), lambda b: (b, 0, 0)),
            whole((D, E)), whole((D, E)), whole((E, R)), whole((E, N)),
            whole((E, N)), whole((R, E)), whole((1, E)), whole((N, E)),
            whole((1, E)), whole((E, D)), whole((1, D)), whole((1, D)),
        ],
        out_specs=pl.BlockSpec((1, L, D), lambda b: (b, 0, 0)),
        scratch_shapes=[
            pltpu.VMEM((L, E), f32),
            pltpu.VMEM((L, E), f32),
            pltpu.VMEM((L, E), f32),
            pltpu.VMEM((L, N), f32),
            pltpu.VMEM((L, N), f32),
            pltpu.VMEM((N, E), f32),
            pltpu.VMEM((L, N, E), f32),
            pltpu.VMEM((L, N, E), f32),
            pltpu.VMEM((L, N, E), f32),
            pltpu.VMEM((L, E), f32),
            pltpu.VMEM((4, N, E), f32),
        ],
        compiler_params=pltpu.CompilerParams(
            dimension_semantics=("parallel",),
            vmem_limit_bytes=56 * 1024 * 1024,
        ),
        name="vssm_quad_scan",
    )(x, w_in_x, w_in_z, wx_dt, wx_b, wx_c, w_dt_t, b_dt.reshape(1, E),
      a_log_t, D_param.reshape(1, E), w_out_t, gamma.reshape(1, D),
      beta.reshape(1, D))


# speedup vs baseline: 17.5560x; 1.0996x over previous
import jax
import jax.numpy as jnp
from jax.experimental import pallas as pl
from jax.experimental.pallas import tpu as pltpu

B, L, D = 8, 256, 384
E, N, R = 768, 16, 24
HW = 16
EPS = 1e-5
FILL_CHUNK = 8
UNROLL = 8
GROUPS = L // UNROLL


def _vssm_kernel(x_ref, w_in_x_ref, w_in_z_ref, wx_dt_ref, wx_b_ref,
                 wx_c_ref, w_dt_t_ref, b_dt_ref, a_log_t_ref, d_param_ref,
                 w_out_t_ref, gamma_ref, beta_ref, o_ref,
                 xin_s, z_s, dt_s, bp_s, cp_s, a_s,
                 abar_s, bbar_s, cful_s, y0_s, y1_s, y2_s, y3_s):
    xb = x_ref[0]

    xin = jnp.dot(xb, w_in_x_ref[...], preferred_element_type=jnp.float32)
    xin_s[...] = xin
    z_s[...] = jnp.dot(xb, w_in_z_ref[...], preferred_element_type=jnp.float32)
    dt_in = jnp.dot(xin, wx_dt_ref[...], preferred_element_type=jnp.float32)
    bp_s[...] = jnp.dot(xin, wx_b_ref[...], preferred_element_type=jnp.float32)
    cp_s[...] = jnp.dot(xin, wx_c_ref[...], preferred_element_type=jnp.float32)
    dt_raw = jnp.dot(dt_in, w_dt_t_ref[...],
                     preferred_element_type=jnp.float32) + b_dt_ref[...]
    dt_s[...] = jnp.maximum(dt_raw, 0.0) + jnp.log1p(jnp.exp(-jnp.abs(dt_raw)))
    a_s[...] = -jnp.exp(a_log_t_ref[...])

    def fill_body(i, _):
        sl = pl.ds(i * FILL_CHUNK, FILL_CHUNK)
        dt_c = dt_s[sl, :]
        xin_c = xin_s[sl, :]
        bp_c = bp_s[sl, :]
        cp_c = cp_s[sl, :]
        a_v = a_s[...]
        abar_s[sl] = jnp.exp(dt_c[:, None, :] * a_v[None, :, :])
        bbar_s[sl] = (dt_c * xin_c)[:, None, :] * bp_c[:, :, None]
        cful_s[sl] = jnp.broadcast_to(cp_c[:, :, None], (FILL_CHUNK, N, E))
        return 0

    jax.lax.fori_loop(0, L // FILL_CHUNK, fill_body, 0)

    def run_dir(y_ref, load_row, store_fwd):
        def body(i, h):
            fwd_base = pl.multiple_of(UNROLL * i, UNROLL)
            rev_base = pl.multiple_of((L - UNROLL) - UNROLL * i, UNROLL)
            for k in range(UNROLL):
                l = load_row(i, k)
                hn = abar_s[l] * h + bbar_s[l]
                red = jnp.sum(hn * cful_s[l], axis=0, keepdims=True)
                if store_fwd:
                    y_ref[pl.ds(fwd_base + k, 1), :] = red
                else:
                    y_ref[pl.ds(rev_base + (UNROLL - 1 - k), 1), :] = red
                h = hn
            return h

        jax.lax.fori_loop(0, GROUPS, body, jnp.zeros((N, E), jnp.float32))

    perm = lambda t: ((t & (HW - 1)) << 4) | (t >> 4)
    run_dir(y0_s, lambda i, k: UNROLL * i + k, True)
    run_dir(y1_s, lambda i, k: (L - 1) - (UNROLL * i + k), False)
    run_dir(y2_s, lambda i, k: perm(UNROLL * i + k), True)
    run_dir(y3_s, lambda i, k: perm((L - 1) - (UNROLL * i + k)), False)

    t2 = jnp.swapaxes(y2_s[...].reshape(HW, HW, E), 0, 1).reshape(L, E)
    t3 = jnp.swapaxes(y3_s[...].reshape(HW, HW, E), 0, 1).reshape(L, E)
    ysum = y0_s[...] + y1_s[...] + t2 + t3

    z = z_s[...]
    sig = 1.0 / (1.0 + jnp.exp(-z))
    y = ysum * 0.25 * (z * sig) + xin_s[...] * d_param_ref[...]
    out = jnp.dot(y, w_out_t_ref[...], preferred_element_type=jnp.float32) + xb
    mu = jnp.mean(out, axis=-1, keepdims=True)
    xc = out - mu
    var = jnp.mean(xc * xc, axis=-1, keepdims=True)
    o_ref[0] = xc * jax.lax.rsqrt(var + EPS) * gamma_ref[...] + beta_ref[...]


@jax.jit
def kernel(x, W_in, A_log, W_x, W_dt, b_dt, D_param, W_out, gamma, beta):
    w_in_x = W_in[:E].T
    w_in_z = W_in[E:].T
    wx_dt = W_x[:R].T
    wx_b = W_x[R:R + N].T
    wx_c = W_x[R + N:].T
    w_dt_t = W_dt.T
    a_log_t = A_log.T
    w_out_t = W_out.T

    whole = lambda s: pl.BlockSpec(s, lambda b: tuple(0 for _ in s))
    f32 = jnp.float32
    return pl.pallas_call(
        _vssm_kernel,
        out_shape=jax.ShapeDtypeStruct((B, L, D), f32),
        grid=(B,),
        in_specs=[
            pl.BlockSpec((1, L, D), lambda b: (b, 0, 0)),
            whole((D, E)), whole((D, E)), whole((E, R)), whole((E, N)),
            whole((E, N)), whole((R, E)), whole((1, E)), whole((N, E)),
            whole((1, E)), whole((E, D)), whole((1, D)), whole((1, D)),
        ],
        out_specs=pl.BlockSpec((1, L, D), lambda b: (b, 0, 0)),
        scratch_shapes=[
            pltpu.VMEM((L, E), f32),
            pltpu.VMEM((L, E), f32),
            pltpu.VMEM((L, E), f32),
            pltpu.VMEM((L, N), f32),
            pltpu.VMEM((L, N), f32),
            pltpu.VMEM((N, E), f32),
            pltpu.VMEM((L, N, E), f32),
            pltpu.VMEM((L, N, E), f32),
            pltpu.VMEM((L, N, E), f32),
            pltpu.VMEM((L, E), f32),
            pltpu.VMEM((L, E), f32),
            pltpu.VMEM((L, E), f32),
            pltpu.VMEM((L, E), f32),
        ],
        compiler_params=pltpu.CompilerParams(
            dimension_semantics=("parallel",),
            vmem_limit_bytes=56 * 1024 * 1024,
        ),
        name="vssm_quad_scan",
    )(x, w_in_x, w_in_z, wx_dt, wx_b, wx_c, w_dt_t, b_dt.reshape(1, E),
      a_log_t, D_param.reshape(1, E), w_out_t, gamma.reshape(1, D),
      beta.reshape(1, D))
